# baseline (speedup 1.0000x reference)
import jax
import jax.numpy as jnp
from jax.experimental import pallas as pl
from jax.experimental.pallas import tpu as pltpu

_DIM = 2048
_N_EXPERTS = 64
_N_GROUPS = 2
_GROUP_SIZE = _N_EXPERTS // _N_GROUPS
_BLOCK = 2048

_KEY_MIN = -2147483647 - 1


def _to_key(s):
    u = jax.lax.bitcast_convert_type(s, jnp.int32)
    return jnp.where(u < 0, u ^ jnp.int32(0x7FFFFFFF), u)


def _from_key(k):
    u = jnp.where(k < 0, k ^ jnp.int32(0x7FFFFFFF), k)
    return jax.lax.bitcast_convert_type(u, jnp.float32)


def _gate_block(x_ref, wt_ref, w_out_ref, i_out_ref):
    s = jnp.dot(x_ref[...], wt_ref[...], preferred_element_type=jnp.float32)

    lane = jax.lax.broadcasted_iota(jnp.int32, s.shape, 1)
    key = _to_key(s)

    km0 = jnp.where(lane < _GROUP_SIZE, key, _KEY_MIN)
    km1 = jnp.where(lane >= _GROUP_SIZE, key, _KEY_MIN)
    kg0 = jnp.max(km0, axis=-1, keepdims=True)
    kg1 = jnp.max(km1, axis=-1, keepdims=True)
    k1 = jnp.maximum(kg0, kg1)
    i1 = jnp.min(jnp.where(key == k1, lane, _N_EXPERTS), axis=-1, keepdims=True)

    drop = lane == i1
    kd0 = jnp.max(jnp.where(drop, _KEY_MIN, km0), axis=-1, keepdims=True)
    kd1 = jnp.max(jnp.where(drop, _KEY_MIN, km1), axis=-1, keepdims=True)
    k2 = jnp.where(kg1 > kg0, kd1, kd0)
    i2 = jnp.min(
        jnp.where(jnp.logical_or(drop, key != k2), _N_EXPERTS, lane),
        axis=-1,
        keepdims=True,
    )

    m = _from_key(k1)
    z = jnp.sum(jnp.exp(s - m), axis=-1, keepdims=True)
    w1 = jnp.exp(_from_key(k1) - m) / z
    w2 = jnp.exp(_from_key(k2) - m) / z

    w_out_ref[...] = jnp.concatenate([w1, w2], axis=-1)
    i_out_ref[...] = jnp.concatenate([i1, i2], axis=-1)


@jax.jit
def kernel(x, router_w):
    n = x.shape[0]
    grid = (n // _BLOCK,)
    wt = router_w.T
    weights, indices = pl.pallas_call(
        _gate_block,
        grid=grid,
        in_specs=[
            pl.BlockSpec((_BLOCK, _DIM), lambda i: (i, 0)),
            pl.BlockSpec((_DIM, _N_EXPERTS), lambda i: (0, 0)),
        ],
        out_specs=[
            pl.BlockSpec((_BLOCK, 2), lambda i: (i, 0)),
            pl.BlockSpec((_BLOCK, 2), lambda i: (i, 0)),
        ],
        out_shape=[
            jax.ShapeDtypeStruct((n, 2), jnp.float32),
            jax.ShapeDtypeStruct((n, 2), jnp.int32),
        ],
        compiler_params=pltpu.CompilerParams(
            dimension_semantics=("arbitrary",),
        ),
    )(x, wt)
    return weights, indices
